# Initial kernel scaffold; baseline (speedup 1.0000x reference)
import jax
import jax.numpy as jnp
from jax.experimental import pallas as pl


def kernel(x, edge_index, edge_weight, Wa0, ba0, Wa1, ba1, Wl0, bl0, Wl1, bl1):
    raise NotImplementedError("write your pallas kernel here")


# speedup vs baseline: 4.1942x; 4.1942x over previous
import functools

import jax
import jax.numpy as jnp
from jax import lax
from jax.experimental import pallas as pl
from jax.experimental.pallas import tpu as pltpu
from jax.experimental.pallas import tpu_sc as plsc

_N = 10000
_D = 128
_NC = 2
_NS = 16
_NW = _NC * _NS
_C = 80
_L = 16

_RPT = 640
_RPT_LAST = _N - 15 * _RPT
_ZB = 80


def _spmm_body(x_hbm, src_hbm, dst_hbm, w_hbm, out_hbm,
               src_v, dst_v, w_v, rows_v, zb_v, acc_sh, sem):
    c = lax.axis_index("c")
    s = lax.axis_index("s")
    wid = s * _NC + c

    nedges = src_hbm.shape[0]
    epw = nedges // _NW
    nchunk = epw // _C

    zero16 = jnp.zeros((_L,), jnp.float32)

    def zrow(i, _):
        for j in range(_D // _L):
            zb_v[i, pl.ds(j * _L, _L)] = zero16
        return 0
    lax.fori_loop(0, _ZB, zrow, 0)

    r0 = s * _RPT
    nz = jnp.where(s < _NS - 1, _RPT // _ZB, _RPT_LAST // _ZB)

    def zcopy(i, _):
        off = pl.multiple_of(r0 + i * _ZB, 8)
        pltpu.sync_copy(zb_v, acc_sh.at[pl.ds(off, _ZB)])
        return 0
    lax.fori_loop(0, nz, zcopy, 0)

    plsc.subcore_barrier()

    base = wid * epw

    def chunk(k, _):
        off = pl.multiple_of(base + k * _C, 8)
        pltpu.sync_copy(src_hbm.at[pl.ds(off, _C)], src_v)
        pltpu.sync_copy(dst_hbm.at[pl.ds(off, _C)], dst_v)
        pltpu.sync_copy(w_hbm.at[pl.ds(off, _C)], w_v)
        pltpu.async_copy(x_hbm.at[src_v], rows_v, sem).wait()

        for g in range(_C // _L):
            wv = w_v[pl.ds(g * _L, _L)]
            for l in range(_L):
                w = wv[l]
                e = g * _L + l
                for j in range(_D // _L):
                    sl = pl.ds(j * _L, _L)
                    rows_v[e, sl] = rows_v[e, sl] * w

        pltpu.sync_copy(rows_v, acc_sh.at[dst_v], add=True)
        return 0
    lax.fori_loop(0, nchunk, chunk, 0)

    plsc.subcore_barrier()
    ro = pl.multiple_of(r0, 8)

    @pl.when(s < _NS - 1)
    def _():
        pltpu.sync_copy(acc_sh.at[pl.ds(ro, _RPT)],
                        out_hbm.at[c, pl.ds(ro, _RPT)])

    @pl.when(s == _NS - 1)
    def _():
        pltpu.sync_copy(acc_sh.at[pl.ds(ro, _RPT_LAST)],
                        out_hbm.at[c, pl.ds(ro, _RPT_LAST)])


def _spmm(x, src, dst, w):
    mesh = plsc.VectorSubcoreMesh(core_axis_name="c", subcore_axis_name="s")
    f = pl.kernel(
        _spmm_body,
        mesh=mesh,
        out_type=jax.ShapeDtypeStruct((_NC, x.shape[0], _D), jnp.float32),
        scratch_types=[
            pltpu.VMEM((_C,), jnp.int32),
            pltpu.VMEM((_C,), jnp.int32),
            pltpu.VMEM((_C,), jnp.float32),
            pltpu.VMEM((_C, _D), jnp.float32),
            pltpu.VMEM((_ZB, _D), jnp.float32),
            pltpu.VMEM_SHARED((x.shape[0], _D), jnp.float32),
            pltpu.SemaphoreType.DMA,
        ],
    )
    return f(x, src, dst, w)


_R = 2000


def _tc1_body(x_ref, p0_ref, p1_ref, wa_ref, ba_ref, wla_ref, wlb_ref,
              bl_ref, h_ref):
    sgm = p0_ref[0] + p1_ref[0]
    agg = jnp.maximum(
        jnp.dot(sgm, wa_ref[...], preferred_element_type=jnp.float32)
        + ba_ref[...], 0.0)
    hv = (jnp.dot(x_ref[...], wla_ref[...], preferred_element_type=jnp.float32)
          + jnp.dot(agg, wlb_ref[...], preferred_element_type=jnp.float32)
          + bl_ref[...])
    h_ref[...] = jnp.maximum(hv, 0.0)


def _tc2_body(h_ref, q0_ref, q1_ref, wa_ref, ba_ref, wla_ref, wlb_ref,
              bl_ref, o_ref):
    sgm = q0_ref[0] + q1_ref[0]
    agg = jnp.maximum(
        jnp.dot(sgm, wa_ref[...], preferred_element_type=jnp.float32)
        + ba_ref[...], 0.0)
    ov = (jnp.dot(h_ref[...], wla_ref[...], preferred_element_type=jnp.float32)
          + jnp.dot(agg, wlb_ref[...], preferred_element_type=jnp.float32)
          + bl_ref[...])
    nrm = jnp.sqrt(jnp.sum(ov * ov, axis=1, keepdims=True))
    o_ref[...] = ov / jnp.maximum(nrm, 1e-12)


def _dense_layer(body, xh, p, Wa, ba, Wl, bl):
    grid = (_N // _R,)
    specs = [
        pl.BlockSpec((_R, _D), lambda i: (i, 0)),
        pl.BlockSpec((1, _R, _D), lambda i: (0, i, 0)),
        pl.BlockSpec((1, _R, _D), lambda i: (1, i, 0)),
        pl.BlockSpec((_D, _D), lambda i: (0, 0)),
        pl.BlockSpec((1, _D), lambda i: (0, 0)),
        pl.BlockSpec((_D, _D), lambda i: (0, 0)),
        pl.BlockSpec((_D, _D), lambda i: (0, 0)),
        pl.BlockSpec((1, _D), lambda i: (0, 0)),
    ]
    return pl.pallas_call(
        body,
        grid=grid,
        in_specs=specs,
        out_specs=pl.BlockSpec((_R, _D), lambda i: (i, 0)),
        out_shape=jax.ShapeDtypeStruct((_N, _D), jnp.float32),
    )(xh, p, p, Wa, ba.reshape(1, _D), Wl[:_D], Wl[_D:], bl.reshape(1, _D))


def kernel(x, edge_index, edge_weight, Wa0, ba0, Wa1, ba1, Wl0, bl0, Wl1, bl1):
    src = edge_index[1].astype(jnp.int32)
    dst = edge_index[0].astype(jnp.int32)
    w = edge_weight.astype(jnp.float32)

    p = _spmm(x, src, dst, w)
    h = _dense_layer(_tc1_body, x, p, Wa0, ba0, Wl0, bl0)
    q = _spmm(h, src, dst, w)
    return _dense_layer(_tc2_body, h, q, Wa1, ba1, Wl1, bl1)
